# baseline (speedup 1.0000x reference)
import jax
import jax.numpy as jnp
from jax import lax
from jax.experimental import pallas as pl
from jax.experimental.pallas import tpu as pltpu
from jax.experimental.pallas import tpu_sc as plsc

_N = 10000
_E = 320000
_C = 128
_NC = 2
_NS = 16
_NW = _NC * _NS
_L = 16
_EPT = _E // _NW
_EPP = 10240
_KA = 64
_NCA = _EPP // _KA
_KC = 128
_NCC = _EPP // _KC
_BLK = 2048
_LBL = 1024
_NBM = _EPP // _LBL
_NPAD = 10240
_RPT = _NPAD // _NS
_NEG = -1e30


def _proj_body(x_ref, wl_ref, wr_ref, xcat_ref):
    x = x_ref[...]
    xcat_ref[pl.ds(0, _N), :] = jnp.dot(
        x, wl_ref[...], preferred_element_type=jnp.float32)
    xcat_ref[pl.ds(_N, _N), :] = jnp.dot(
        x, wr_ref[...], preferred_element_type=jnp.float32)


def _proj(x, W_l, W_r):
    return pl.pallas_call(
        _proj_body,
        out_shape=jax.ShapeDtypeStruct((2 * _N, _C), jnp.float32),
    )(x, W_l, W_r)


def _sc_body(xcat_hbm, att_hbm, srcpk_hbm, dnpk_hbm,
             m_out, d_out, s_out, l_hbm, m_stage,
             md, rboth, srcb, dnb, lbc, gidx, sb,
             mro, mbuf0, mbuf1, tbuf, att_vm, s_sh, sem):
    cid = lax.axis_index("c")
    sid = lax.axis_index("s")
    wid = cid * _NS + sid
    ibase = wid * _EPP
    lbase = wid * _EPP

    pltpu.sync_copy(att_hbm, att_vm)
    att_s = [att_vm[pl.ds(f * _L, _L)] for f in range(_C // _L)]
    iota16 = lax.iota(jnp.int32, _L)
    iota_row = iota16 * _L

    def _init(i, _):
        md[pl.ds(i * _L, _L)] = jnp.full((_L,), _NEG, jnp.float32)
        return 0
    lax.fori_loop(0, _NPAD // _L, _init, 0)

    def _chunk_a(c, _):
        @pl.when(c % (_BLK // _KA) == 0)
        def _():
            boff = ibase + (c // (_BLK // _KA)) * _BLK
            pltpu.sync_copy(srcpk_hbm.at[pl.ds(boff, _BLK)], srcb)
            pltpu.sync_copy(dnpk_hbm.at[pl.ds(boff, _BLK)], dnb)

        j = (c % (_BLK // _KA)) * _KA
        for h in range(_KA // _L):
            gidx[pl.ds(h * _L, _L)] = srcb[pl.ds(j + h * _L, _L)]
            gidx[pl.ds(_KA + h * _L, _L)] = dnb[pl.ds(j + h * _L, _L)]
        pltpu.async_copy(xcat_hbm.at[gidx], rboth, sem).wait()

        lo = (c % (_LBL // _KA)) * _KA

        def _group(g, _):
            e0 = g * _L
            for i in range(_L):
                acc = None
                for f in range(_C // _L):
                    v = rboth[e0 + i, pl.ds(f * _L, _L)] \
                        + rboth[_KA + e0 + i, pl.ds(f * _L, _L)]
                    lr = 0.6 * v + 0.4 * jnp.abs(v)
                    t = lr * att_s[f]
                    acc = t if acc is None else acc + t
                tbuf[pl.ds(i * _L, _L)] = acc
            tot = None
            for j16 in range(_L):
                col = plsc.load_gather(tbuf, [iota_row + j16])
                tot = col if tot is None else tot + col
            lbc[pl.ds(lo + e0, _L)] = tot
            d16 = gidx[pl.ds(_KA + e0, _L)] - _N
            cur = plsc.load_gather(md, [d16])
            plsc.store_scatter(md, [d16], jnp.maximum(cur, tot))
            return 0
        lax.fori_loop(0, _KA // _L, _group, 0)

        @pl.when(c % (_LBL // _KA) == (_LBL // _KA) - 1)
        def _():
            pltpu.sync_copy(
                lbc,
                l_hbm.at[pl.ds(lbase + (c - (_LBL // _KA - 1)) * _KA, _LBL)])
        return 0
    lax.fori_loop(0, _NCA, _chunk_a, 0)

    pltpu.sync_copy(md, m_stage.at[pl.ds(wid * _NPAD, _NPAD)])
    plsc.subcore_barrier()
    rbase = sid * _RPT
    sbase = cid * _NS * _NPAD + rbase
    bufs = (mbuf0, mbuf1)
    for t in (0, 1):
        pltpu.async_copy(m_stage.at[pl.ds(sbase + t * _NPAD, _RPT)],
                         bufs[t % 2], sem)
    for t in range(_NS):
        pltpu.make_async_copy(m_stage.at[pl.ds(sbase + t * _NPAD, _RPT)],
                              bufs[t % 2], sem).wait()
        if t + 2 < _NS:
            pltpu.async_copy(m_stage.at[pl.ds(sbase + (t + 2) * _NPAD, _RPT)],
                             bufs[t % 2], sem)

        def _red(i, _, _t=t):
            v = bufs[_t % 2][pl.ds(i * _L, _L)]
            if _t == 0:
                mro[pl.ds(i * _L, _L)] = v
            else:
                mro[pl.ds(i * _L, _L)] = jnp.maximum(mro[pl.ds(i * _L, _L)], v)
            return 0
        lax.fori_loop(0, _RPT // _L, _red, 0)
    pltpu.sync_copy(mro, m_out.at[pl.ds(cid * _NPAD + rbase, _RPT)])
    plsc.subcore_barrier()
    pltpu.sync_copy(m_out.at[pl.ds(cid * _NPAD, _NPAD)], md)

    def _batch_m(b, _):
        @pl.when(b % 2 == 0)
        def _():
            pltpu.sync_copy(
                dnpk_hbm.at[pl.ds(ibase + (b // 2) * _BLK, _BLK)], dnb)
        pltpu.sync_copy(l_hbm.at[pl.ds(lbase + b * _LBL, _LBL)], lbc)
        jo = (b % 2) * _LBL

        def _mgroup(g, _):
            e0 = g * _L
            d16 = dnb[pl.ds(jo + e0, _L)] - _N
            l16 = lbc[pl.ds(e0, _L)]
            m16 = plsc.load_gather(md, [d16])
            u16 = jnp.exp(l16 - m16)
            mask = (b * _LBL + e0 + iota16) < _EPT
            lbc[pl.ds(e0, _L)] = jnp.where(mask, u16, 0.0)
            return 0
        lax.fori_loop(0, _LBL // _L, _mgroup, 0)
        pltpu.sync_copy(lbc, l_hbm.at[pl.ds(lbase + b * _LBL, _LBL)])
        return 0
    lax.fori_loop(0, _NBM, _batch_m, 0)

    def _initd(i, _):
        md[pl.ds(i * _L, _L)] = jnp.zeros((_L,), jnp.float32)
        return 0
    lax.fori_loop(0, _NPAD // _L, _initd, 0)

    def _z(i, _):
        for f in range(_C // _L):
            rboth[i, pl.ds(f * _L, _L)] = jnp.zeros((_L,), jnp.float32)
        return 0
    lax.fori_loop(0, _KC, _z, 0)

    def _z2(k, _):
        pltpu.sync_copy(rboth, s_sh.at[pl.ds(rbase + k * _KC, _KC)])
        return 0
    lax.fori_loop(0, _RPT // _KC, _z2, 0)
    plsc.subcore_barrier()

    def _chunk_c(c, _):
        @pl.when(c % (_BLK // _KC) == 0)
        def _():
            boff = ibase + (c // (_BLK // _KC)) * _BLK
            pltpu.sync_copy(srcpk_hbm.at[pl.ds(boff, _BLK)], srcb)
            pltpu.sync_copy(dnpk_hbm.at[pl.ds(boff, _BLK)], dnb)

        @pl.when(c % (_LBL // _KC) == 0)
        def _():
            pltpu.sync_copy(l_hbm.at[pl.ds(lbase + c * _KC, _LBL)], lbc)

        j = (c % (_BLK // _KC)) * _KC
        for h in range(_KC // _L):
            sb[pl.ds(h * _L, _L)] = dnb[pl.ds(j + h * _L, _L)] - _N
        pltpu.async_copy(
            xcat_hbm.at[srcb.at[pl.ds(j, _KC)]], rboth, sem).wait()

        lo = (c % (_LBL // _KC)) * _KC

        def _group(g, _):
            e0 = g * _L
            d16 = sb[pl.ds(e0, _L)]
            u16 = lbc[pl.ds(lo + e0, _L)]
            plsc.addupdate_scatter(md, [d16], u16)
            for i in range(_L):
                u = u16[i]
                for f in range(_C // _L):
                    rboth[e0 + i, pl.ds(f * _L, _L)] = \
                        rboth[e0 + i, pl.ds(f * _L, _L)] * u
            return 0
        lax.fori_loop(0, _KC // _L, _group, 0)
        pltpu.sync_copy(rboth, s_sh.at[sb], add=True)
        return 0
    lax.fori_loop(0, _NCC, _chunk_c, 0)

    pltpu.sync_copy(md, d_out.at[pl.ds(wid * _NPAD, _NPAD)])
    plsc.subcore_barrier()
    pltpu.sync_copy(s_sh.at[pl.ds(rbase, _RPT)],
                    s_out.at[pl.ds(cid * _NPAD + rbase, _RPT)])


def _sc_call(xcat, att_v, srcpk, dnpk):
    outs = pl.kernel(
        _sc_body,
        out_type=[
            jax.ShapeDtypeStruct((_NC * _NPAD,), jnp.float32),
            jax.ShapeDtypeStruct((_NC * _NS * _NPAD,), jnp.float32),
            jax.ShapeDtypeStruct((_NC * _NPAD, _C), jnp.float32),
            jax.ShapeDtypeStruct((_NW * _EPP,), jnp.float32),
            jax.ShapeDtypeStruct((_NC * _NS * _NPAD,), jnp.float32),
        ],
        mesh=plsc.VectorSubcoreMesh(core_axis_name="c", subcore_axis_name="s"),
        compiler_params=pltpu.CompilerParams(needs_layout_passes=False),
        scratch_types=[
            pltpu.VMEM((_NPAD,), jnp.float32),
            pltpu.VMEM((_KC, _C), jnp.float32),
            pltpu.VMEM((_BLK,), jnp.int32),
            pltpu.VMEM((_BLK,), jnp.int32),
            pltpu.VMEM((_LBL,), jnp.float32),
            pltpu.VMEM((2 * _KA,), jnp.int32),
            pltpu.VMEM((_KC,), jnp.int32),
            pltpu.VMEM((_RPT,), jnp.float32),
            pltpu.VMEM((_RPT,), jnp.float32),
            pltpu.VMEM((_RPT,), jnp.float32),
            pltpu.VMEM((_L * _L,), jnp.float32),
            pltpu.VMEM((_C,), jnp.float32),
            pltpu.VMEM_SHARED((_NPAD, _C), jnp.float32),
            pltpu.SemaphoreType.DMA,
        ],
    )(xcat, att_v, srcpk, dnpk)
    return (outs[0].reshape(_NC, _NPAD),
            outs[1].reshape(_NC, _NS, _NPAD),
            outs[2].reshape(_NC, _NPAD, _C))


def _merge_body(m_ref, d_ref, s_ref, bias_ref, lnw_ref, lnb_ref, out_ref):
    m = m_ref[...]
    mm = jnp.max(m, axis=0, keepdims=True)
    w = jnp.exp(m - mm)
    dsum = jnp.sum(d_ref[...], axis=1)
    den = jnp.sum(dsum * w, axis=0)
    s = jnp.sum(s_ref[...] * w[:, :, None], axis=0)
    pre = s / (den[:, None] + 1e-16) + bias_ref[...][None, :]
    pre = pre[:_N]
    mu = jnp.mean(pre)
    xc = pre - mu
    var = jnp.mean(xc * xc)
    out_ref[...] = xc * lax.rsqrt(var + 1e-5) * lnw_ref[...][None, :] \
        + lnb_ref[...][None, :]


def _merge(m_p, d_p, s_p, bias, ln_weight, ln_bias):
    return pl.pallas_call(
        _merge_body,
        out_shape=jax.ShapeDtypeStruct((_N, _C), jnp.float32),
    )(m_p, d_p, s_p, bias, ln_weight, ln_bias)


def kernel(x, edge_index, W_l, W_r, att, bias, ln_weight, ln_bias):
    xcat = _proj(x, W_l, W_r)
    att_v = att.reshape(_C)
    pad = jnp.zeros((_NW, _EPP - _EPT), jnp.int32)
    srcpk = jnp.concatenate(
        [edge_index[0].reshape(_NW, _EPT), pad], axis=1).reshape(-1)
    dnpk = jnp.concatenate(
        [edge_index[1].reshape(_NW, _EPT) + _N, pad + _N], axis=1).reshape(-1)
    m_p, d_p, s_p = _sc_call(xcat, att_v, srcpk, dnpk)
    return _merge(m_p, d_p, s_p, bias, ln_weight, ln_bias)


# speedup vs baseline: 1.5258x; 1.5258x over previous
import jax
import jax.numpy as jnp
from jax import lax
from jax.experimental import pallas as pl
from jax.experimental.pallas import tpu as pltpu
from jax.experimental.pallas import tpu_sc as plsc

_N = 10000
_E = 320000
_C = 128
_NC = 2
_NS = 16
_NW = _NC * _NS
_L = 16
_EPT = _E // _NW
_K = 80
_NCHUNK = _EPT // _K
_NPAD = 10240
_RPT = _NPAD // _NS
_NEG = -1e30


def _proj_body(x_ref, wl_ref, wr_ref, xl_ref, xr_ref):
    x = x_ref[...]
    xl_ref[...] = jnp.dot(x, wl_ref[...], preferred_element_type=jnp.float32)
    xr_ref[...] = jnp.dot(x, wr_ref[...], preferred_element_type=jnp.float32)


def _proj(x, W_l, W_r):
    return pl.pallas_call(
        _proj_body,
        out_shape=[
            jax.ShapeDtypeStruct((_N, _C), jnp.float32),
            jax.ShapeDtypeStruct((_N, _C), jnp.float32),
        ],
    )(x, W_l, W_r)


def _sc_body(xl_hbm, xr_hbm, att_hbm, epk_hbm,
             m_out, d_out, s_out, l_hbm, m_stage,
             m_loc, d_loc, rl, rr, mro, mbuf0, mbuf1, lblk, eblk, sb,
             tbuf, att_vm, s_sh, sem):
    cid = lax.axis_index("c")
    sid = lax.axis_index("s")
    wid = cid * _NS + sid
    ebase = wid * _NCHUNK * (2 * _K)
    lbase = wid * _EPT

    pltpu.sync_copy(att_hbm, att_vm)
    att_s = [att_vm[pl.ds(f * _L, _L)] for f in range(_C // _L)]

    def _init(i, _):
        m_loc[pl.ds(i * _L, _L)] = jnp.full((_L,), _NEG, jnp.float32)
        d_loc[pl.ds(i * _L, _L)] = jnp.zeros((_L,), jnp.float32)
        return 0
    lax.fori_loop(0, _NPAD // _L, _init, 0)

    iota_row = lax.iota(jnp.int32, _L) * _L

    def _chunk_a(c, _):
        @pl.when(c % 5 == 0)
        def _():
            pltpu.sync_copy(
                epk_hbm.at[pl.ds(ebase + (c // 5) * (10 * _K), 10 * _K)],
                eblk)

        j = (c % 5) * (2 * _K)
        pltpu.async_copy(xl_hbm.at[eblk.at[pl.ds(j, _K)]], rl, sem).wait()
        pltpu.async_copy(xr_hbm.at[eblk.at[pl.ds(j + _K, _K)]], rr,
                         sem).wait()
        lo = (c % 5) * _K

        def _group(g, _):
            e0 = g * _L
            for i in range(_L):
                acc = None
                for f in range(_C // _L):
                    v = rl[e0 + i, pl.ds(f * _L, _L)] \
                        + rr[e0 + i, pl.ds(f * _L, _L)]
                    lr = 0.6 * v + 0.4 * jnp.abs(v)
                    t = lr * att_s[f]
                    acc = t if acc is None else acc + t
                tbuf[pl.ds(i * _L, _L)] = acc
            tot = None
            for jj in range(_L):
                col = plsc.load_gather(tbuf, [iota_row + jj])
                tot = col if tot is None else tot + col
            lblk[pl.ds(lo + e0, _L)] = tot
            d16 = eblk[pl.ds(j + _K + e0, _L)]
            cur = plsc.load_gather(m_loc, [d16])
            plsc.store_scatter(m_loc, [d16], jnp.maximum(cur, tot))
            return 0
        lax.fori_loop(0, _K // _L, _group, 0)

        @pl.when(c % 5 == 4)
        def _():
            pltpu.sync_copy(
                lblk, l_hbm.at[pl.ds(lbase + (c - 4) * _K, 5 * _K)])
        return 0
    lax.fori_loop(0, _NCHUNK, _chunk_a, 0)

    pltpu.sync_copy(m_loc, m_stage.at[pl.ds(wid * _NPAD, _NPAD)])
    plsc.subcore_barrier()
    rbase = sid * _RPT
    bufs = [mbuf0, mbuf1]
    sbase = cid * _NS * _NPAD + rbase
    cps = [pltpu.async_copy(m_stage.at[pl.ds(sbase + t * _NPAD, _RPT)],
                            bufs[t % 2], sem) for t in (0, 1)]
    for t in range(_NS):
        cps[t % 2].wait()
        if t + 2 < _NS:
            cps[t % 2] = pltpu.async_copy(
                m_stage.at[pl.ds(sbase + (t + 2) * _NPAD, _RPT)],
                bufs[t % 2], sem)

        def _red(i, _, _t=t):
            v = bufs[_t % 2][pl.ds(i * _L, _L)]
            if _t == 0:
                mro[pl.ds(i * _L, _L)] = v
            else:
                mro[pl.ds(i * _L, _L)] = jnp.maximum(mro[pl.ds(i * _L, _L)], v)
            return 0
        lax.fori_loop(0, _RPT // _L, _red, 0)
    pltpu.sync_copy(mro, m_out.at[pl.ds(cid * _NPAD + rbase, _RPT)])
    plsc.subcore_barrier()
    pltpu.sync_copy(m_out.at[pl.ds(cid * _NPAD, _NPAD)], m_loc)

    def _z(i, _):
        for f in range(_C // _L):
            rl[i, pl.ds(f * _L, _L)] = jnp.zeros((_L,), jnp.float32)
        return 0
    lax.fori_loop(0, _K, _z, 0)

    def _z2(k, _):
        pltpu.sync_copy(rl, s_sh.at[pl.ds(rbase + k * _K, _K)])
        return 0
    lax.fori_loop(0, _RPT // _K, _z2, 0)
    plsc.subcore_barrier()

    def _pc(c, _):
        @pl.when(c % 5 == 0)
        def _():
            pltpu.sync_copy(
                epk_hbm.at[pl.ds(ebase + (c // 5) * (10 * _K), 10 * _K)],
                eblk)
            pltpu.sync_copy(l_hbm.at[pl.ds(lbase + c * _K, 5 * _K)], lblk)

        j = (c % 5) * (2 * _K)
        lo = (c % 5) * _K
        pltpu.async_copy(xl_hbm.at[eblk.at[pl.ds(j, _K)]], rl, sem).wait()
        for h in range(_K // _L):
            sb[pl.ds(h * _L, _L)] = eblk[pl.ds(j + _K + h * _L, _L)]

        def _group(g, _):
            e0 = g * _L
            d16 = sb[pl.ds(e0, _L)]
            l16 = lblk[pl.ds(lo + e0, _L)]
            m16 = plsc.load_gather(m_loc, [d16])
            u16 = jnp.exp(l16 - m16)
            plsc.addupdate_scatter(d_loc, [d16], u16)
            for i in range(_L):
                u = u16[i]
                for f in range(_C // _L):
                    rl[e0 + i, pl.ds(f * _L, _L)] = \
                        rl[e0 + i, pl.ds(f * _L, _L)] * u
            return 0
        lax.fori_loop(0, _K // _L, _group, 0)
        pltpu.sync_copy(rl, s_sh.at[sb], add=True)
        return 0
    lax.fori_loop(0, _NCHUNK, _pc, 0)
    pltpu.sync_copy(d_loc, d_out.at[pl.ds(wid * _NPAD, _NPAD)])
    plsc.subcore_barrier()
    pltpu.sync_copy(s_sh.at[pl.ds(rbase, _RPT)],
                    s_out.at[pl.ds(cid * _NPAD + rbase, _RPT)])


def _sc_call(xl, xr, att_v, epk):
    outs = pl.kernel(
        _sc_body,
        out_type=[
            jax.ShapeDtypeStruct((_NC * _NPAD,), jnp.float32),
            jax.ShapeDtypeStruct((_NC * _NS * _NPAD,), jnp.float32),
            jax.ShapeDtypeStruct((_NC * _NPAD, _C), jnp.float32),
            jax.ShapeDtypeStruct((_E,), jnp.float32),
            jax.ShapeDtypeStruct((_NC * _NS * _NPAD,), jnp.float32),
        ],
        mesh=plsc.VectorSubcoreMesh(core_axis_name="c", subcore_axis_name="s"),
        compiler_params=pltpu.CompilerParams(needs_layout_passes=False),
        scratch_types=[
            pltpu.VMEM((_NPAD,), jnp.float32),
            pltpu.VMEM((_NPAD,), jnp.float32),
            pltpu.VMEM((_K, _C), jnp.float32),
            pltpu.VMEM((_K, _C), jnp.float32),
            pltpu.VMEM((_RPT,), jnp.float32),
            pltpu.VMEM((_RPT,), jnp.float32),
            pltpu.VMEM((_RPT,), jnp.float32),
            pltpu.VMEM((5 * _K,), jnp.float32),
            pltpu.VMEM((5 * 2 * _K,), jnp.int32),
            pltpu.VMEM((_K,), jnp.int32),
            pltpu.VMEM((_L * _L,), jnp.float32),
            pltpu.VMEM((_C,), jnp.float32),
            pltpu.VMEM_SHARED((_NPAD, _C), jnp.float32),
            pltpu.SemaphoreType.DMA,
        ],
    )(xl, xr, att_v, epk)
    return (outs[0].reshape(_NC, _NPAD),
            outs[1].reshape(_NC, _NS, _NPAD),
            outs[2].reshape(_NC, _NPAD, _C))


def _merge_body(m_ref, d_ref, s_ref, bias_ref, lnw_ref, lnb_ref, out_ref):
    m = m_ref[...]
    mm = jnp.max(m, axis=0, keepdims=True)
    w = jnp.exp(m - mm)
    dsum = jnp.sum(d_ref[...], axis=1)
    den = jnp.sum(dsum * w, axis=0)
    s = jnp.sum(s_ref[...] * w[:, :, None], axis=0)
    pre = s / (den[:, None] + 1e-16) + bias_ref[...][None, :]
    pre = pre[:_N]
    mu = jnp.mean(pre)
    xc = pre - mu
    var = jnp.mean(xc * xc)
    out_ref[...] = xc * lax.rsqrt(var + 1e-5) * lnw_ref[...][None, :] \
        + lnb_ref[...][None, :]


def _merge(m_p, d_p, s_p, bias, ln_weight, ln_bias):
    return pl.pallas_call(
        _merge_body,
        out_shape=jax.ShapeDtypeStruct((_N, _C), jnp.float32),
    )(m_p, d_p, s_p, bias, ln_weight, ln_bias)


def kernel(x, edge_index, W_l, W_r, att, bias, ln_weight, ln_bias):
    xl, xr = _proj(x, W_l, W_r)
    att_v = att.reshape(_C)
    epk = jnp.concatenate(
        [edge_index[0].reshape(_NW, _NCHUNK, _K),
         edge_index[1].reshape(_NW, _NCHUNK, _K)], axis=2).reshape(-1)
    m_p, d_p, s_p = _sc_call(xl, xr, att_v, epk)
    return _merge(m_p, d_p, s_p, bias, ln_weight, ln_bias)
